# baseline (speedup 1.0000x reference)
import functools

import jax
import jax.numpy as jnp
from jax import lax
from jax.experimental import pallas as pl
from jax.experimental.pallas import tpu as pltpu
from jax.experimental.pallas import tpu_sc as plsc

N = 10000
E = 320000
DF = 128
DE = 16
H = 128
G = 64

EB = 2000
NB = 2000


def _node_mm_body(x_ref, w1a_ref, w3a_ref, xa_ref, xw3_ref):
    x = x_ref[...]
    xa_ref[...] = jnp.dot(x, w1a_ref[...], preferred_element_type=jnp.float32)
    xw3_ref[...] = jnp.dot(x, w3a_ref[...], preferred_element_type=jnp.float32)


def _node_mm(x, w1a, w3a):
    return pl.pallas_call(
        _node_mm_body,
        grid=(N // NB,),
        in_specs=[
            pl.BlockSpec((NB, DF), lambda i: (i, 0)),
            pl.BlockSpec((DF, H), lambda i: (0, 0)),
            pl.BlockSpec((DF, H), lambda i: (0, 0)),
        ],
        out_specs=[
            pl.BlockSpec((NB, H), lambda i: (i, 0)),
            pl.BlockSpec((NB, H), lambda i: (i, 0)),
        ],
        out_shape=[
            jax.ShapeDtypeStruct((N, H), jnp.float32),
            jax.ShapeDtypeStruct((N, H), jnp.float32),
        ],
    )(x, w1a, w3a)


def _edge_init_body(gx_ref, ea_ref, w1b_ref, w2_ref, h0_ref, hw_ref):
    h0 = jax.nn.relu(
        gx_ref[...]
        + jnp.dot(ea_ref[...], w1b_ref[...], preferred_element_type=jnp.float32)
    )
    h0_ref[...] = h0
    hw_ref[...] = jnp.dot(h0, w2_ref[...], preferred_element_type=jnp.float32)


def _edge_init(gx, ea, w1b, w2):
    return pl.pallas_call(
        _edge_init_body,
        grid=(E // EB,),
        in_specs=[
            pl.BlockSpec((EB, H), lambda i: (i, 0)),
            pl.BlockSpec((EB, DE), lambda i: (i, 0)),
            pl.BlockSpec((DE, H), lambda i: (0, 0)),
            pl.BlockSpec((H, H), lambda i: (0, 0)),
        ],
        out_specs=[
            pl.BlockSpec((EB, H), lambda i: (i, 0)),
            pl.BlockSpec((EB, H), lambda i: (i, 0)),
        ],
        out_shape=[
            jax.ShapeDtypeStruct((E, H), jnp.float32),
            jax.ShapeDtypeStruct((E, H), jnp.float32),
        ],
    )(gx, ea, w1b, w2)


def _pair_swap(hw):
    rows = jax.lax.broadcasted_iota(jnp.int32, hw.shape, 0)
    even = (rows % 2) == 0
    return jnp.where(even, jnp.roll(hw, -1, axis=0), jnp.roll(hw, 1, axis=0))


def _edge_round_body(h0_ref, g_ref, hw_ref, w2_ref, hwn_ref):
    h = jax.nn.relu(h0_ref[...] + g_ref[...] - _pair_swap(hw_ref[...]))
    hwn_ref[...] = jnp.dot(h, w2_ref[...], preferred_element_type=jnp.float32)


def _edge_round(h0, g, hw, w2):
    return pl.pallas_call(
        _edge_round_body,
        grid=(E // EB,),
        in_specs=[
            pl.BlockSpec((EB, H), lambda i: (i, 0)),
            pl.BlockSpec((EB, H), lambda i: (i, 0)),
            pl.BlockSpec((EB, H), lambda i: (i, 0)),
            pl.BlockSpec((H, H), lambda i: (0, 0)),
        ],
        out_specs=pl.BlockSpec((EB, H), lambda i: (i, 0)),
        out_shape=jax.ShapeDtypeStruct((E, H), jnp.float32),
    )(h0, g, hw, w2)


def _edge_final_body(h0_ref, g_ref, hw_ref, h_ref):
    h_ref[...] = jax.nn.relu(h0_ref[...] + g_ref[...] - _pair_swap(hw_ref[...]))


def _edge_final(h0, g, hw):
    return pl.pallas_call(
        _edge_final_body,
        grid=(E // EB,),
        in_specs=[
            pl.BlockSpec((EB, H), lambda i: (i, 0)),
            pl.BlockSpec((EB, H), lambda i: (i, 0)),
            pl.BlockSpec((EB, H), lambda i: (i, 0)),
        ],
        out_specs=pl.BlockSpec((EB, H), lambda i: (i, 0)),
        out_shape=jax.ShapeDtypeStruct((E, H), jnp.float32),
    )(h0, g, hw)


def _node_out_body(xw3_ref, vm_ref, w3b_ref, bcol_ref, out_ref):
    i = pl.program_id(0)

    @pl.when(i == 0)
    def _():
        out_ref[...] = jnp.zeros_like(out_ref)

    na = jax.nn.relu(
        xw3_ref[...]
        + jnp.dot(vm_ref[...], w3b_ref[...], preferred_element_type=jnp.float32)
    )
    gids = jax.lax.broadcasted_iota(jnp.int32, (NB, G), 1)
    onehot = (bcol_ref[...] == gids).astype(jnp.float32)
    out_ref[...] += jax.lax.dot_general(
        onehot, na, (((0,), (0,)), ((), ())), preferred_element_type=jnp.float32
    )


def _node_out(xw3, vmsg, w3b, bcol):
    return pl.pallas_call(
        _node_out_body,
        grid=(N // NB,),
        in_specs=[
            pl.BlockSpec((NB, H), lambda i: (i, 0)),
            pl.BlockSpec((NB, H), lambda i: (i, 0)),
            pl.BlockSpec((H, H), lambda i: (0, 0)),
            pl.BlockSpec((NB, 1), lambda i: (i, 0)),
        ],
        out_specs=pl.BlockSpec((G, H), lambda i: (0, 0)),
        out_shape=jax.ShapeDtypeStruct((G, H), jnp.float32),
        compiler_params=pltpu.CompilerParams(
            dimension_semantics=("arbitrary",)
        ),
    )(xw3, vmsg, w3b, bcol)


def _add2_body(a_ref, b_ref, o_ref):
    o_ref[...] = a_ref[...] + b_ref[...]


def _add2(p):
    return pl.pallas_call(
        _add2_body,
        grid=(N // NB,),
        in_specs=[
            pl.BlockSpec((NB, H), lambda i: (i, 0)),
            pl.BlockSpec((NB, H), lambda i: (i, 0)),
        ],
        out_specs=pl.BlockSpec((NB, H), lambda i: (i, 0)),
        out_shape=jax.ShapeDtypeStruct((N, H), jnp.float32),
    )(p[0], p[1])


SC_CORES = 2
SC_TILES = 16
NW = SC_CORES * SC_TILES
CHUNK = 128
ECH = E // CHUNK
NCH = 80
ECH_PAD = NCH * NW
EIDX_PAD = ECH_PAD * CHUNK
NPT = 632
NPT_LAST = N - 15 * NPT

GNB = 5
SNB = 2
GPAD = ((NCH + GNB - 1) // GNB) * GNB
SPAD = ((NCH + SNB - 1) // SNB) * SNB


def _sc_mesh():
    return plsc.VectorSubcoreMesh(core_axis_name="c", subcore_axis_name="s")


def _sc_gather(table, idx):
    @functools.partial(
        pl.kernel,
        mesh=_sc_mesh(),
        out_type=jax.ShapeDtypeStruct((ECH, CHUNK, H), jnp.float32),
        scratch_types=[
            pltpu.VMEM((NCH, CHUNK), jnp.int32),
            pltpu.VMEM((GNB, CHUNK, H), jnp.float32),
            pltpu.SemaphoreType.DMA((GNB,)),
        ],
    )
    def k(table_hbm, idx_hbm, out_hbm, idx_v, rows_v, gsem):
        wid = lax.axis_index("s") * SC_CORES + lax.axis_index("c")
        pltpu.sync_copy(idx_hbm.at[wid], idx_v)

        @pl.loop(0, GPAD, step=GNB)
        def _(jj):
            rbase = wid * NCH + jj

            for b in range(GNB):
                j = jj + b
                r = wid * NCH + j

                @pl.when(r < ECH)
                def _():
                    pltpu.async_copy(
                        table_hbm.at[idx_v.at[j]], rows_v.at[b], gsem.at[b]
                    )

            for b in range(GNB):
                j = jj + b
                r = wid * NCH + j

                @pl.when(r < ECH)
                def _():
                    pltpu.make_async_copy(
                        table_hbm.at[idx_v.at[j]], rows_v.at[b], gsem.at[b]
                    ).wait()

            @pl.when(rbase + GNB <= ECH)
            def _():
                pltpu.sync_copy(rows_v, out_hbm.at[pl.ds(rbase, GNB)])

            @pl.when((rbase < ECH) & (rbase + GNB > ECH))
            def _():
                for b in range(GNB):
                    r = rbase + b

                    @pl.when(r < ECH)
                    def _():
                        pltpu.sync_copy(rows_v.at[b], out_hbm.at[r])

    return k(table, idx).reshape(E, H)


def _sc_segsum_partials(rows, idx, zeros):
    @functools.partial(
        pl.kernel,
        mesh=_sc_mesh(),
        out_type=jax.ShapeDtypeStruct((SC_CORES, N, H), jnp.float32),
        scratch_types=[
            pltpu.VMEM((NCH, CHUNK), jnp.int32),
            pltpu.VMEM((SNB, CHUNK, H), jnp.float32),
            pltpu.SemaphoreType.DMA((SNB,)),
            pltpu.VMEM_SHARED((N, H), jnp.float32),
        ],
    )
    def k(rows_hbm, idx_hbm, zeros_hbm, out_hbm, idx_v, rows_v, rsem, acc):
        cid = lax.axis_index("c")
        sid = lax.axis_index("s")
        wid = sid * SC_CORES + cid
        nbase = pl.multiple_of(sid * NPT, 8)
        pltpu.sync_copy(idx_hbm.at[wid], idx_v)

        @pl.when(sid < 15)
        def _():
            pltpu.sync_copy(
                zeros_hbm.at[pl.ds(nbase, NPT)], acc.at[pl.ds(nbase, NPT)]
            )

        @pl.when(sid == 15)
        def _():
            pltpu.sync_copy(
                zeros_hbm.at[pl.ds(15 * NPT, NPT_LAST)],
                acc.at[pl.ds(15 * NPT, NPT_LAST)],
            )

        plsc.subcore_barrier()

        @pl.loop(0, SPAD, step=SNB)
        def _(jj):
            for b in range(SNB):
                j = jj + b
                r = wid * NCH + j

                @pl.when(r < ECH)
                def _():
                    base = pl.multiple_of(r * CHUNK, CHUNK)
                    pltpu.async_copy(
                        rows_hbm.at[pl.ds(base, CHUNK)], rows_v.at[b], rsem.at[b]
                    )

            for b in range(SNB):
                j = jj + b
                r = wid * NCH + j

                @pl.when(r < ECH)
                def _():
                    base = pl.multiple_of(r * CHUNK, CHUNK)
                    pltpu.make_async_copy(
                        rows_hbm.at[pl.ds(base, CHUNK)], rows_v.at[b], rsem.at[b]
                    ).wait()
                    pltpu.sync_copy(rows_v.at[b], acc.at[idx_v.at[j]], add=True)

        plsc.subcore_barrier()

        @pl.when(sid < 15)
        def _():
            pltpu.sync_copy(
                acc.at[pl.ds(nbase, NPT)], out_hbm.at[cid].at[pl.ds(nbase, NPT)]
            )

        @pl.when(sid == 15)
        def _():
            pltpu.sync_copy(
                acc.at[pl.ds(15 * NPT, NPT_LAST)],
                out_hbm.at[cid].at[pl.ds(15 * NPT, NPT_LAST)],
            )

    return k(rows, idx, zeros)


def _gather_rows(table, idx):
    return _sc_gather(table, idx)


def _segsum(rows, dst, zeros):
    return _add2(_sc_segsum_partials(rows, dst, zeros))


def kernel(x, edge_index, edge_attr, batch, W1, W2, W3):
    def _arrange(ix):
        ixp = jnp.concatenate(
            [ix.astype(jnp.int32), jnp.zeros((EIDX_PAD - E,), jnp.int32)]
        )
        return ixp.reshape(NW, NCH, CHUNK)

    src1 = _arrange(edge_index[0])
    dst1 = _arrange(edge_index[1])
    w1a, w1b = W1[:DF], W1[DF:]
    w3a, w3b = W3[:DF], W3[DF:]
    bcol = batch.astype(jnp.int32).reshape(N, 1)
    zeros = jnp.zeros((N, H), jnp.float32)

    xa, xw3 = _node_mm(x, w1a, w3a)
    gx = _gather_rows(xa, src1)
    h0, hw0 = _edge_init(gx, edge_attr, w1b, W2)

    agg1 = _segsum(hw0, dst1, zeros)
    g1 = _gather_rows(agg1, src1)
    hw1 = _edge_round(h0, g1, hw0, W2)

    agg2 = _segsum(hw1, dst1, zeros)
    g2 = _gather_rows(agg2, src1)
    h2 = _edge_final(h0, g2, hw1)

    vmsg = _segsum(h2, dst1, zeros)
    return _node_out(xw3, vmsg, w3b, bcol)


# speedup vs baseline: 1.0538x; 1.0538x over previous
import functools

import jax
import jax.numpy as jnp
from jax import lax
from jax.experimental import pallas as pl
from jax.experimental.pallas import tpu as pltpu
from jax.experimental.pallas import tpu_sc as plsc

N = 10000
E = 320000
DF = 128
DE = 16
H = 128
G = 64

EB = 2000
NB = 2000


def _node_mm_body(x_ref, w1a_ref, w3a_ref, xa_ref, xw3_ref):
    x = x_ref[...]
    xa_ref[...] = jnp.dot(x, w1a_ref[...], preferred_element_type=jnp.float32)
    xw3_ref[...] = jnp.dot(x, w3a_ref[...], preferred_element_type=jnp.float32)


def _node_mm(x, w1a, w3a):
    return pl.pallas_call(
        _node_mm_body,
        grid=(N // NB,),
        in_specs=[
            pl.BlockSpec((NB, DF), lambda i: (i, 0)),
            pl.BlockSpec((DF, H), lambda i: (0, 0)),
            pl.BlockSpec((DF, H), lambda i: (0, 0)),
        ],
        out_specs=[
            pl.BlockSpec((NB, H), lambda i: (i, 0)),
            pl.BlockSpec((NB, H), lambda i: (i, 0)),
        ],
        out_shape=[
            jax.ShapeDtypeStruct((N, H), jnp.float32),
            jax.ShapeDtypeStruct((N, H), jnp.float32),
        ],
    )(x, w1a, w3a)


def _edge_init_body(gx_ref, ea_ref, w1b_ref, w2_ref, h0_ref, hw_ref):
    h0 = jax.nn.relu(
        gx_ref[...]
        + jnp.dot(ea_ref[...], w1b_ref[...], preferred_element_type=jnp.float32)
    )
    h0_ref[...] = h0
    hw_ref[...] = jnp.dot(
        h0.astype(jnp.bfloat16),
        w2_ref[...].astype(jnp.bfloat16),
        preferred_element_type=jnp.float32,
    )


def _edge_init(gx, ea, w1b, w2):
    return pl.pallas_call(
        _edge_init_body,
        grid=(E // EB,),
        in_specs=[
            pl.BlockSpec((EB, H), lambda i: (i, 0)),
            pl.BlockSpec((EB, DE), lambda i: (i, 0)),
            pl.BlockSpec((DE, H), lambda i: (0, 0)),
            pl.BlockSpec((H, H), lambda i: (0, 0)),
        ],
        out_specs=[
            pl.BlockSpec((EB, H), lambda i: (i, 0)),
            pl.BlockSpec((EB, H), lambda i: (i, 0)),
        ],
        out_shape=[
            jax.ShapeDtypeStruct((E, H), jnp.float32),
            jax.ShapeDtypeStruct((E, H), jnp.float32),
        ],
    )(gx, ea, w1b, w2)


def _pair_swap(hw):
    rows = jax.lax.broadcasted_iota(jnp.int32, hw.shape, 0)
    even = (rows % 2) == 0
    return jnp.where(even, jnp.roll(hw, -1, axis=0), jnp.roll(hw, 1, axis=0))


def _edge_round_body(h0_ref, g_ref, hw_ref, w2_ref, hwn_ref):
    h = jax.nn.relu(h0_ref[...] + g_ref[...] - _pair_swap(hw_ref[...]))
    hwn_ref[...] = jnp.dot(
        h.astype(jnp.bfloat16),
        w2_ref[...].astype(jnp.bfloat16),
        preferred_element_type=jnp.float32,
    )


def _edge_round(h0, g, hw, w2):
    return pl.pallas_call(
        _edge_round_body,
        grid=(E // EB,),
        in_specs=[
            pl.BlockSpec((EB, H), lambda i: (i, 0)),
            pl.BlockSpec((EB, H), lambda i: (i, 0)),
            pl.BlockSpec((EB, H), lambda i: (i, 0)),
            pl.BlockSpec((H, H), lambda i: (0, 0)),
        ],
        out_specs=pl.BlockSpec((EB, H), lambda i: (i, 0)),
        out_shape=jax.ShapeDtypeStruct((E, H), jnp.float32),
    )(h0, g, hw, w2)


def _edge_final_body(h0_ref, g_ref, hw_ref, h_ref):
    h_ref[...] = jax.nn.relu(h0_ref[...] + g_ref[...] - _pair_swap(hw_ref[...]))


def _edge_final(h0, g, hw):
    return pl.pallas_call(
        _edge_final_body,
        grid=(E // EB,),
        in_specs=[
            pl.BlockSpec((EB, H), lambda i: (i, 0)),
            pl.BlockSpec((EB, H), lambda i: (i, 0)),
            pl.BlockSpec((EB, H), lambda i: (i, 0)),
        ],
        out_specs=pl.BlockSpec((EB, H), lambda i: (i, 0)),
        out_shape=jax.ShapeDtypeStruct((E, H), jnp.float32),
    )(h0, g, hw)


def _node_out_body(xw3_ref, vm_ref, w3b_ref, bcol_ref, out_ref):
    i = pl.program_id(0)

    @pl.when(i == 0)
    def _():
        out_ref[...] = jnp.zeros_like(out_ref)

    na = jax.nn.relu(
        xw3_ref[...]
        + jnp.dot(vm_ref[...], w3b_ref[...], preferred_element_type=jnp.float32)
    )
    gids = jax.lax.broadcasted_iota(jnp.int32, (NB, G), 1)
    onehot = (bcol_ref[...] == gids).astype(jnp.float32)
    out_ref[...] += jax.lax.dot_general(
        onehot, na, (((0,), (0,)), ((), ())), preferred_element_type=jnp.float32
    )


def _node_out(xw3, vmsg, w3b, bcol):
    return pl.pallas_call(
        _node_out_body,
        grid=(N // NB,),
        in_specs=[
            pl.BlockSpec((NB, H), lambda i: (i, 0)),
            pl.BlockSpec((NB, H), lambda i: (i, 0)),
            pl.BlockSpec((H, H), lambda i: (0, 0)),
            pl.BlockSpec((NB, 1), lambda i: (i, 0)),
        ],
        out_specs=pl.BlockSpec((G, H), lambda i: (0, 0)),
        out_shape=jax.ShapeDtypeStruct((G, H), jnp.float32),
        compiler_params=pltpu.CompilerParams(
            dimension_semantics=("arbitrary",)
        ),
    )(xw3, vmsg, w3b, bcol)


def _add2_body(a_ref, b_ref, o_ref):
    o_ref[...] = a_ref[...] + b_ref[...]


def _add2(p):
    return pl.pallas_call(
        _add2_body,
        grid=(N // NB,),
        in_specs=[
            pl.BlockSpec((NB, H), lambda i: (i, 0)),
            pl.BlockSpec((NB, H), lambda i: (i, 0)),
        ],
        out_specs=pl.BlockSpec((NB, H), lambda i: (i, 0)),
        out_shape=jax.ShapeDtypeStruct((N, H), jnp.float32),
    )(p[0], p[1])


SC_CORES = 2
SC_TILES = 16
NW = SC_CORES * SC_TILES
CHUNK = 128
ECH = E // CHUNK
NCH = 80
ECH_PAD = NCH * NW
EIDX_PAD = ECH_PAD * CHUNK
NPT = 632
NPT_LAST = N - 15 * NPT

GNB = 6
SNB = 2
GPAD = ((NCH + GNB - 1) // GNB) * GNB
SPAD = ((NCH + SNB - 1) // SNB) * SNB


def _sc_mesh():
    return plsc.VectorSubcoreMesh(core_axis_name="c", subcore_axis_name="s")


def _sc_gather(table, idx):
    @functools.partial(
        pl.kernel,
        mesh=_sc_mesh(),
        out_type=jax.ShapeDtypeStruct((E, H), jnp.float32),
        scratch_types=[
            pltpu.VMEM((NCH, CHUNK), jnp.int32),
            pltpu.VMEM((GNB, CHUNK, H), jnp.float32),
            pltpu.SemaphoreType.DMA((GNB,)),
            pltpu.SemaphoreType.DMA((GNB,)),
        ],
    )
    def k(table_hbm, idx_hbm, out_hbm, idx_v, rows_v, gsem, osem):
        wid = lax.axis_index("s") * SC_CORES + lax.axis_index("c")
        pltpu.sync_copy(idx_hbm.at[wid], idx_v)

        @pl.loop(0, GPAD, step=GNB)
        def _(jj):
            for b in range(GNB):
                j = jj + b
                r = wid + NW * j

                @pl.when(r < ECH)
                def _():
                    pltpu.async_copy(
                        table_hbm.at[idx_v.at[j]], rows_v.at[b], gsem.at[b]
                    )

            for b in range(GNB):
                j = jj + b
                r = wid + NW * j

                @pl.when(r < ECH)
                def _():
                    pltpu.make_async_copy(
                        table_hbm.at[idx_v.at[j]], rows_v.at[b], gsem.at[b]
                    ).wait()
                    base = pl.multiple_of(r * CHUNK, CHUNK)
                    pltpu.async_copy(
                        rows_v.at[b], out_hbm.at[pl.ds(base, CHUNK)], osem.at[b]
                    )

            for b in range(GNB):
                j = jj + b
                r = wid + NW * j

                @pl.when(r < ECH)
                def _():
                    base = pl.multiple_of(r * CHUNK, CHUNK)
                    pltpu.make_async_copy(
                        rows_v.at[b], out_hbm.at[pl.ds(base, CHUNK)], osem.at[b]
                    ).wait()

    return k(table, idx)


def _sc_segsum_partials(rows, idx, zeros):
    @functools.partial(
        pl.kernel,
        mesh=_sc_mesh(),
        out_type=jax.ShapeDtypeStruct((SC_CORES, N, H), jnp.float32),
        scratch_types=[
            pltpu.VMEM((NCH, CHUNK), jnp.int32),
            pltpu.VMEM((SNB, CHUNK, H), jnp.float32),
            pltpu.SemaphoreType.DMA((SNB,)),
            pltpu.VMEM_SHARED((N, H), jnp.float32),
        ],
    )
    def k(rows_hbm, idx_hbm, zeros_hbm, out_hbm, idx_v, rows_v, rsem, acc):
        cid = lax.axis_index("c")
        sid = lax.axis_index("s")
        wid = sid * SC_CORES + cid
        nbase = pl.multiple_of(sid * NPT, 8)
        pltpu.sync_copy(idx_hbm.at[wid], idx_v)

        @pl.when(sid < 15)
        def _():
            pltpu.sync_copy(
                zeros_hbm.at[pl.ds(nbase, NPT)], acc.at[pl.ds(nbase, NPT)]
            )

        @pl.when(sid == 15)
        def _():
            pltpu.sync_copy(
                zeros_hbm.at[pl.ds(15 * NPT, NPT_LAST)],
                acc.at[pl.ds(15 * NPT, NPT_LAST)],
            )

        plsc.subcore_barrier()

        @pl.loop(0, SPAD, step=SNB)
        def _(jj):
            for b in range(SNB):
                j = jj + b
                r = wid + NW * j

                @pl.when(r < ECH)
                def _():
                    base = pl.multiple_of(r * CHUNK, CHUNK)
                    pltpu.async_copy(
                        rows_hbm.at[pl.ds(base, CHUNK)], rows_v.at[b], rsem.at[b]
                    )

            for b in range(SNB):
                j = jj + b
                r = wid + NW * j

                @pl.when(r < ECH)
                def _():
                    base = pl.multiple_of(r * CHUNK, CHUNK)
                    pltpu.make_async_copy(
                        rows_hbm.at[pl.ds(base, CHUNK)], rows_v.at[b], rsem.at[b]
                    ).wait()
                    pltpu.sync_copy(rows_v.at[b], acc.at[idx_v.at[j]], add=True)

        plsc.subcore_barrier()

        @pl.when(sid < 15)
        def _():
            pltpu.sync_copy(
                acc.at[pl.ds(nbase, NPT)], out_hbm.at[cid].at[pl.ds(nbase, NPT)]
            )

        @pl.when(sid == 15)
        def _():
            pltpu.sync_copy(
                acc.at[pl.ds(15 * NPT, NPT_LAST)],
                out_hbm.at[cid].at[pl.ds(15 * NPT, NPT_LAST)],
            )

    return k(rows, idx, zeros)


def _gather_rows(table, idx):
    return _sc_gather(table, idx)


def _segsum(rows, dst, zeros):
    return _add2(_sc_segsum_partials(rows, dst, zeros))


def kernel(x, edge_index, edge_attr, batch, W1, W2, W3):
    def _arrange(ix):
        ixp = jnp.concatenate(
            [ix.astype(jnp.int32), jnp.zeros((EIDX_PAD - E,), jnp.int32)]
        )
        return ixp.reshape(NCH, NW, CHUNK).transpose(1, 0, 2)

    src1 = _arrange(edge_index[0])
    dst1 = _arrange(edge_index[1])
    w1a, w1b = W1[:DF], W1[DF:]
    w3a, w3b = W3[:DF], W3[DF:]
    bcol = batch.astype(jnp.int32).reshape(N, 1)
    zeros = jnp.zeros((N, H), jnp.float32)

    xa, xw3 = _node_mm(x, w1a, w3a)
    gx = _gather_rows(xa, src1)
    h0, hw0 = _edge_init(gx, edge_attr, w1b, W2)

    agg1 = _segsum(hw0, dst1, zeros)
    g1 = _gather_rows(agg1, src1)
    hw1 = _edge_round(h0, g1, hw0, W2)

    agg2 = _segsum(hw1, dst1, zeros)
    g2 = _gather_rows(agg2, src1)
    h2 = _edge_final(h0, g2, hw1)

    vmsg = _segsum(h2, dst1, zeros)
    return _node_out(xw3, vmsg, w3b, bcol)


# speedup vs baseline: 1.0750x; 1.0201x over previous
import functools

import jax
import jax.numpy as jnp
from jax import lax
from jax.experimental import pallas as pl
from jax.experimental.pallas import tpu as pltpu
from jax.experimental.pallas import tpu_sc as plsc

N = 10000
E = 320000
DF = 128
DE = 16
H = 128
G = 64

EB = 2000
NB = 2000


def _node_mm_body(x_ref, w1a_ref, w3a_ref, xa_ref, xw3_ref):
    x = x_ref[...]
    xa_ref[...] = jnp.dot(x, w1a_ref[...], preferred_element_type=jnp.float32)
    xw3_ref[...] = jnp.dot(x, w3a_ref[...], preferred_element_type=jnp.float32)


def _node_mm(x, w1a, w3a):
    return pl.pallas_call(
        _node_mm_body,
        grid=(N // NB,),
        in_specs=[
            pl.BlockSpec((NB, DF), lambda i: (i, 0)),
            pl.BlockSpec((DF, H), lambda i: (0, 0)),
            pl.BlockSpec((DF, H), lambda i: (0, 0)),
        ],
        out_specs=[
            pl.BlockSpec((NB, H), lambda i: (i, 0)),
            pl.BlockSpec((NB, H), lambda i: (i, 0)),
        ],
        out_shape=[
            jax.ShapeDtypeStruct((N, H), jnp.float32),
            jax.ShapeDtypeStruct((N, H), jnp.float32),
        ],
    )(x, w1a, w3a)


def _edge_init_body(gx_ref, ea_ref, w1b_ref, w2_ref, h0_ref, hw_ref):
    h0 = jax.nn.relu(
        gx_ref[...]
        + jnp.dot(ea_ref[...], w1b_ref[...], preferred_element_type=jnp.float32)
    )
    h0_ref[...] = h0.astype(jnp.bfloat16)
    hw_ref[...] = jnp.dot(
        h0.astype(jnp.bfloat16),
        w2_ref[...].astype(jnp.bfloat16),
        preferred_element_type=jnp.float32,
    )


def _edge_init(gx, ea, w1b, w2):
    return pl.pallas_call(
        _edge_init_body,
        grid=(E // EB,),
        in_specs=[
            pl.BlockSpec((EB, H), lambda i: (i, 0)),
            pl.BlockSpec((EB, DE), lambda i: (i, 0)),
            pl.BlockSpec((DE, H), lambda i: (0, 0)),
            pl.BlockSpec((H, H), lambda i: (0, 0)),
        ],
        out_specs=[
            pl.BlockSpec((EB, H), lambda i: (i, 0)),
            pl.BlockSpec((EB, H), lambda i: (i, 0)),
        ],
        out_shape=[
            jax.ShapeDtypeStruct((E, H), jnp.bfloat16),
            jax.ShapeDtypeStruct((E, H), jnp.float32),
        ],
    )(gx, ea, w1b, w2)


def _pair_swap(hw):
    rows = jax.lax.broadcasted_iota(jnp.int32, hw.shape, 0)
    even = (rows % 2) == 0
    return jnp.where(even, jnp.roll(hw, -1, axis=0), jnp.roll(hw, 1, axis=0))


def _edge_round_body(h0_ref, g_ref, hw_ref, w2_ref, hwn_ref):
    h = jax.nn.relu(
        h0_ref[...].astype(jnp.float32) + g_ref[...] - _pair_swap(hw_ref[...])
    )
    hwn_ref[...] = jnp.dot(
        h.astype(jnp.bfloat16),
        w2_ref[...].astype(jnp.bfloat16),
        preferred_element_type=jnp.float32,
    )


def _edge_round(h0, g, hw, w2):
    return pl.pallas_call(
        _edge_round_body,
        grid=(E // EB,),
        in_specs=[
            pl.BlockSpec((EB, H), lambda i: (i, 0)),
            pl.BlockSpec((EB, H), lambda i: (i, 0)),
            pl.BlockSpec((EB, H), lambda i: (i, 0)),
            pl.BlockSpec((H, H), lambda i: (0, 0)),
        ],
        out_specs=pl.BlockSpec((EB, H), lambda i: (i, 0)),
        out_shape=jax.ShapeDtypeStruct((E, H), jnp.float32),
    )(h0, g, hw, w2)


def _edge_final_body(h0_ref, g_ref, hw_ref, h_ref):
    h_ref[...] = jax.nn.relu(
        h0_ref[...].astype(jnp.float32) + g_ref[...] - _pair_swap(hw_ref[...])
    )


def _edge_final(h0, g, hw):
    return pl.pallas_call(
        _edge_final_body,
        grid=(E // EB,),
        in_specs=[
            pl.BlockSpec((EB, H), lambda i: (i, 0)),
            pl.BlockSpec((EB, H), lambda i: (i, 0)),
            pl.BlockSpec((EB, H), lambda i: (i, 0)),
        ],
        out_specs=pl.BlockSpec((EB, H), lambda i: (i, 0)),
        out_shape=jax.ShapeDtypeStruct((E, H), jnp.float32),
    )(h0, g, hw)


def _node_out_body(xw3_ref, vm_ref, w3b_ref, bcol_ref, out_ref):
    i = pl.program_id(0)

    @pl.when(i == 0)
    def _():
        out_ref[...] = jnp.zeros_like(out_ref)

    na = jax.nn.relu(
        xw3_ref[...]
        + jnp.dot(vm_ref[...], w3b_ref[...], preferred_element_type=jnp.float32)
    )
    gids = jax.lax.broadcasted_iota(jnp.int32, (NB, G), 1)
    onehot = (bcol_ref[...] == gids).astype(jnp.float32)
    out_ref[...] += jax.lax.dot_general(
        onehot, na, (((0,), (0,)), ((), ())), preferred_element_type=jnp.float32
    )


def _node_out(xw3, vmsg, w3b, bcol):
    return pl.pallas_call(
        _node_out_body,
        grid=(N // NB,),
        in_specs=[
            pl.BlockSpec((NB, H), lambda i: (i, 0)),
            pl.BlockSpec((NB, H), lambda i: (i, 0)),
            pl.BlockSpec((H, H), lambda i: (0, 0)),
            pl.BlockSpec((NB, 1), lambda i: (i, 0)),
        ],
        out_specs=pl.BlockSpec((G, H), lambda i: (0, 0)),
        out_shape=jax.ShapeDtypeStruct((G, H), jnp.float32),
        compiler_params=pltpu.CompilerParams(
            dimension_semantics=("arbitrary",)
        ),
    )(xw3, vmsg, w3b, bcol)


def _add2_body(a_ref, b_ref, o_ref):
    o_ref[...] = a_ref[...] + b_ref[...]


def _add2(p):
    return pl.pallas_call(
        _add2_body,
        grid=(N // NB,),
        in_specs=[
            pl.BlockSpec((NB, H), lambda i: (i, 0)),
            pl.BlockSpec((NB, H), lambda i: (i, 0)),
        ],
        out_specs=pl.BlockSpec((NB, H), lambda i: (i, 0)),
        out_shape=jax.ShapeDtypeStruct((N, H), jnp.float32),
    )(p[0], p[1])


SC_CORES = 2
SC_TILES = 16
NW = SC_CORES * SC_TILES
CHUNK = 128
ECH = E // CHUNK
NCH = 80
ECH_PAD = NCH * NW
EIDX_PAD = ECH_PAD * CHUNK
NPT = 632
NPT_LAST = N - 15 * NPT

GNB = 6
SNB = 2
GPAD = ((NCH + GNB - 1) // GNB) * GNB
SPAD = ((NCH + SNB - 1) // SNB) * SNB


def _sc_mesh():
    return plsc.VectorSubcoreMesh(core_axis_name="c", subcore_axis_name="s")


def _sc_gather(table, idx):
    @functools.partial(
        pl.kernel,
        mesh=_sc_mesh(),
        out_type=jax.ShapeDtypeStruct((E, H), jnp.float32),
        scratch_types=[
            pltpu.VMEM((NCH, CHUNK), jnp.int32),
            pltpu.VMEM((GNB, CHUNK, H), jnp.float32),
            pltpu.SemaphoreType.DMA((GNB,)),
            pltpu.SemaphoreType.DMA((GNB,)),
        ],
    )
    def k(table_hbm, idx_hbm, out_hbm, idx_v, rows_v, gsem, osem):
        wid = lax.axis_index("s") * SC_CORES + lax.axis_index("c")
        pltpu.sync_copy(idx_hbm.at[wid], idx_v)

        @pl.loop(0, GPAD, step=GNB)
        def _(jj):
            for b in range(GNB):
                j = jj + b
                r = wid + NW * j

                @pl.when(r < ECH)
                def _():
                    pltpu.async_copy(
                        table_hbm.at[idx_v.at[j]], rows_v.at[b], gsem.at[b]
                    )

            for b in range(GNB):
                j = jj + b
                r = wid + NW * j

                @pl.when(r < ECH)
                def _():
                    pltpu.make_async_copy(
                        table_hbm.at[idx_v.at[j]], rows_v.at[b], gsem.at[b]
                    ).wait()
                    base = pl.multiple_of(r * CHUNK, CHUNK)
                    pltpu.async_copy(
                        rows_v.at[b], out_hbm.at[pl.ds(base, CHUNK)], osem.at[b]
                    )

            for b in range(GNB):
                j = jj + b
                r = wid + NW * j

                @pl.when(r < ECH)
                def _():
                    base = pl.multiple_of(r * CHUNK, CHUNK)
                    pltpu.make_async_copy(
                        rows_v.at[b], out_hbm.at[pl.ds(base, CHUNK)], osem.at[b]
                    ).wait()

    return k(table, idx)


def _sc_segsum_partials(rows, idx, zeros):
    @functools.partial(
        pl.kernel,
        mesh=_sc_mesh(),
        out_type=jax.ShapeDtypeStruct((SC_CORES, N, H), jnp.float32),
        scratch_types=[
            pltpu.VMEM((NCH, CHUNK), jnp.int32),
            pltpu.VMEM((SNB, CHUNK, H), jnp.float32),
            pltpu.SemaphoreType.DMA((SNB,)),
            pltpu.VMEM_SHARED((N, H), jnp.float32),
        ],
    )
    def k(rows_hbm, idx_hbm, zeros_hbm, out_hbm, idx_v, rows_v, rsem, acc):
        cid = lax.axis_index("c")
        sid = lax.axis_index("s")
        wid = sid * SC_CORES + cid
        nbase = pl.multiple_of(sid * NPT, 8)
        pltpu.sync_copy(idx_hbm.at[wid], idx_v)

        @pl.when(sid < 15)
        def _():
            pltpu.sync_copy(
                zeros_hbm.at[pl.ds(nbase, NPT)], acc.at[pl.ds(nbase, NPT)]
            )

        @pl.when(sid == 15)
        def _():
            pltpu.sync_copy(
                zeros_hbm.at[pl.ds(15 * NPT, NPT_LAST)],
                acc.at[pl.ds(15 * NPT, NPT_LAST)],
            )

        plsc.subcore_barrier()

        @pl.loop(0, SPAD, step=SNB)
        def _(jj):
            for b in range(SNB):
                j = jj + b
                r = wid + NW * j

                @pl.when(r < ECH)
                def _():
                    base = pl.multiple_of(r * CHUNK, CHUNK)
                    pltpu.async_copy(
                        rows_hbm.at[pl.ds(base, CHUNK)], rows_v.at[b], rsem.at[b]
                    )

            for b in range(SNB):
                j = jj + b
                r = wid + NW * j

                @pl.when(r < ECH)
                def _():
                    base = pl.multiple_of(r * CHUNK, CHUNK)
                    pltpu.make_async_copy(
                        rows_hbm.at[pl.ds(base, CHUNK)], rows_v.at[b], rsem.at[b]
                    ).wait()
                    pltpu.sync_copy(rows_v.at[b], acc.at[idx_v.at[j]], add=True)

        plsc.subcore_barrier()

        @pl.when(sid < 15)
        def _():
            pltpu.sync_copy(
                acc.at[pl.ds(nbase, NPT)], out_hbm.at[cid].at[pl.ds(nbase, NPT)]
            )

        @pl.when(sid == 15)
        def _():
            pltpu.sync_copy(
                acc.at[pl.ds(15 * NPT, NPT_LAST)],
                out_hbm.at[cid].at[pl.ds(15 * NPT, NPT_LAST)],
            )

    return k(rows, idx, zeros)


def _gather_rows(table, idx):
    return _sc_gather(table, idx)


def _segsum(rows, dst, zeros):
    return _add2(_sc_segsum_partials(rows, dst, zeros))


def kernel(x, edge_index, edge_attr, batch, W1, W2, W3):
    def _arrange(ix):
        ixp = jnp.concatenate(
            [ix.astype(jnp.int32), jnp.zeros((EIDX_PAD - E,), jnp.int32)]
        )
        return ixp.reshape(NCH, NW, CHUNK).transpose(1, 0, 2)

    src1 = _arrange(edge_index[0])
    dst1 = _arrange(edge_index[1])
    w1a, w1b = W1[:DF], W1[DF:]
    w3a, w3b = W3[:DF], W3[DF:]
    bcol = batch.astype(jnp.int32).reshape(N, 1)
    zeros = jnp.zeros((N, H), jnp.float32)

    xa, xw3 = _node_mm(x, w1a, w3a)
    gx = _gather_rows(xa, src1)
    h0, hw0 = _edge_init(gx, edge_attr, w1b, W2)

    agg1 = _segsum(hw0, dst1, zeros)
    g1 = _gather_rows(agg1, src1)
    hw1 = _edge_round(h0, g1, hw0, W2)

    agg2 = _segsum(hw1, dst1, zeros)
    g2 = _gather_rows(agg2, src1)
    h2 = _edge_final(h0, g2, hw1)

    vmsg = _segsum(h2, dst1, zeros)
    return _node_out(xw3, vmsg, w3b, bcol)


# speedup vs baseline: 1.0837x; 1.0081x over previous
import functools

import jax
import jax.numpy as jnp
from jax import lax
from jax.experimental import pallas as pl
from jax.experimental.pallas import tpu as pltpu
from jax.experimental.pallas import tpu_sc as plsc

N = 10000
E = 320000
DF = 128
DE = 16
H = 128
G = 64

EB = 2000
NB = 2000


def _node_mm_body(x_ref, w1a_ref, w3a_ref, xa_ref, xw3_ref):
    x = x_ref[...]
    xa_ref[...] = jnp.dot(x, w1a_ref[...], preferred_element_type=jnp.float32)
    xw3_ref[...] = jnp.dot(x, w3a_ref[...], preferred_element_type=jnp.float32)


def _node_mm(x, w1a, w3a):
    return pl.pallas_call(
        _node_mm_body,
        grid=(N // NB,),
        in_specs=[
            pl.BlockSpec((NB, DF), lambda i: (i, 0)),
            pl.BlockSpec((DF, H), lambda i: (0, 0)),
            pl.BlockSpec((DF, H), lambda i: (0, 0)),
        ],
        out_specs=[
            pl.BlockSpec((NB, H), lambda i: (i, 0)),
            pl.BlockSpec((NB, H), lambda i: (i, 0)),
        ],
        out_shape=[
            jax.ShapeDtypeStruct((N, H), jnp.float32),
            jax.ShapeDtypeStruct((N, H), jnp.float32),
        ],
    )(x, w1a, w3a)


def _edge_init_body(gx_ref, ea_ref, w1b_ref, w2_ref, h0_ref, hw_ref):
    h0 = jax.nn.relu(
        gx_ref[...]
        + jnp.dot(ea_ref[...], w1b_ref[...], preferred_element_type=jnp.float32)
    )
    h0_ref[...] = h0.astype(jnp.bfloat16)
    hw_ref[...] = jnp.dot(
        h0.astype(jnp.bfloat16),
        w2_ref[...].astype(jnp.bfloat16),
        preferred_element_type=jnp.float32,
    )


def _edge_init(gx, ea, w1b, w2):
    return pl.pallas_call(
        _edge_init_body,
        grid=(E // EB,),
        in_specs=[
            pl.BlockSpec((EB, H), lambda i: (i, 0)),
            pl.BlockSpec((EB, DE), lambda i: (i, 0)),
            pl.BlockSpec((DE, H), lambda i: (0, 0)),
            pl.BlockSpec((H, H), lambda i: (0, 0)),
        ],
        out_specs=[
            pl.BlockSpec((EB, H), lambda i: (i, 0)),
            pl.BlockSpec((EB, H), lambda i: (i, 0)),
        ],
        out_shape=[
            jax.ShapeDtypeStruct((E, H), jnp.bfloat16),
            jax.ShapeDtypeStruct((E, H), jnp.float32),
        ],
    )(gx, ea, w1b, w2)


def _pair_swap(hw):
    rows = jax.lax.broadcasted_iota(jnp.int32, hw.shape, 0)
    even = (rows % 2) == 0
    return jnp.where(even, jnp.roll(hw, -1, axis=0), jnp.roll(hw, 1, axis=0))


def _edge_round_body(h0_ref, g_ref, hw_ref, w2_ref, hwn_ref):
    h = jax.nn.relu(
        h0_ref[...].astype(jnp.float32) + g_ref[...] - _pair_swap(hw_ref[...])
    )
    hwn_ref[...] = jnp.dot(
        h.astype(jnp.bfloat16),
        w2_ref[...].astype(jnp.bfloat16),
        preferred_element_type=jnp.float32,
    )


def _edge_round(h0, g, hw, w2):
    return pl.pallas_call(
        _edge_round_body,
        grid=(E // EB,),
        in_specs=[
            pl.BlockSpec((EB, H), lambda i: (i, 0)),
            pl.BlockSpec((EB, H), lambda i: (i, 0)),
            pl.BlockSpec((EB, H), lambda i: (i, 0)),
            pl.BlockSpec((H, H), lambda i: (0, 0)),
        ],
        out_specs=pl.BlockSpec((EB, H), lambda i: (i, 0)),
        out_shape=jax.ShapeDtypeStruct((E, H), jnp.float32),
    )(h0, g, hw, w2)


def _edge_final_body(h0_ref, g_ref, hw_ref, h_ref):
    h_ref[...] = jax.nn.relu(
        h0_ref[...].astype(jnp.float32) + g_ref[...] - _pair_swap(hw_ref[...])
    )


def _edge_final(h0, g, hw):
    return pl.pallas_call(
        _edge_final_body,
        grid=(E // EB,),
        in_specs=[
            pl.BlockSpec((EB, H), lambda i: (i, 0)),
            pl.BlockSpec((EB, H), lambda i: (i, 0)),
            pl.BlockSpec((EB, H), lambda i: (i, 0)),
        ],
        out_specs=pl.BlockSpec((EB, H), lambda i: (i, 0)),
        out_shape=jax.ShapeDtypeStruct((E, H), jnp.float32),
    )(h0, g, hw)


def _node_out_body(xw3_ref, vm_ref, w3b_ref, bcol_ref, out_ref):
    i = pl.program_id(0)

    @pl.when(i == 0)
    def _():
        out_ref[...] = jnp.zeros_like(out_ref)

    na = jax.nn.relu(
        xw3_ref[...]
        + jnp.dot(vm_ref[...], w3b_ref[...], preferred_element_type=jnp.float32)
    )
    gids = jax.lax.broadcasted_iota(jnp.int32, (NB, G), 1)
    onehot = (bcol_ref[...] == gids).astype(jnp.float32)
    out_ref[...] += jax.lax.dot_general(
        onehot, na, (((0,), (0,)), ((), ())), preferred_element_type=jnp.float32
    )


def _node_out(xw3, vmsg, w3b, bcol):
    return pl.pallas_call(
        _node_out_body,
        grid=(N // NB,),
        in_specs=[
            pl.BlockSpec((NB, H), lambda i: (i, 0)),
            pl.BlockSpec((NB, H), lambda i: (i, 0)),
            pl.BlockSpec((H, H), lambda i: (0, 0)),
            pl.BlockSpec((NB, 1), lambda i: (i, 0)),
        ],
        out_specs=pl.BlockSpec((G, H), lambda i: (0, 0)),
        out_shape=jax.ShapeDtypeStruct((G, H), jnp.float32),
        compiler_params=pltpu.CompilerParams(
            dimension_semantics=("arbitrary",)
        ),
    )(xw3, vmsg, w3b, bcol)


def _add2_body(a_ref, b_ref, o_ref):
    o_ref[...] = a_ref[...] + b_ref[...]


def _add2(p):
    return pl.pallas_call(
        _add2_body,
        grid=(N // NB,),
        in_specs=[
            pl.BlockSpec((NB, H), lambda i: (i, 0)),
            pl.BlockSpec((NB, H), lambda i: (i, 0)),
        ],
        out_specs=pl.BlockSpec((NB, H), lambda i: (i, 0)),
        out_shape=jax.ShapeDtypeStruct((N, H), jnp.float32),
    )(p[0], p[1])


SC_CORES = 2
SC_TILES = 16
NW = SC_CORES * SC_TILES
CHUNK = 128
ECH = E // CHUNK
NCH = 80
ECH_PAD = NCH * NW
EIDX_PAD = ECH_PAD * CHUNK
NPT = 632
NPT_LAST = N - 15 * NPT

GNB = 7
SNB = 2
GPAD = ((NCH + GNB - 1) // GNB) * GNB
SPAD = ((NCH + SNB - 1) // SNB) * SNB


def _sc_mesh():
    return plsc.VectorSubcoreMesh(core_axis_name="c", subcore_axis_name="s")


def _sc_gather(table, idx):
    @functools.partial(
        pl.kernel,
        mesh=_sc_mesh(),
        out_type=jax.ShapeDtypeStruct((E, H), jnp.float32),
        scratch_types=[
            pltpu.VMEM((NCH, CHUNK), jnp.int32),
            pltpu.VMEM((GNB, CHUNK, H), jnp.float32),
            pltpu.SemaphoreType.DMA((GNB,)),
            pltpu.SemaphoreType.DMA((GNB,)),
        ],
    )
    def k(table_hbm, idx_hbm, out_hbm, idx_v, rows_v, gsem, osem):
        wid = lax.axis_index("s") * SC_CORES + lax.axis_index("c")
        pltpu.sync_copy(idx_hbm.at[wid], idx_v)

        @pl.loop(0, GPAD, step=GNB)
        def _(jj):
            for b in range(GNB):
                j = jj + b
                r = wid + NW * j

                @pl.when(r < ECH)
                def _():
                    pltpu.async_copy(
                        table_hbm.at[idx_v.at[j]], rows_v.at[b], gsem.at[b]
                    )

            for b in range(GNB):
                j = jj + b
                r = wid + NW * j

                @pl.when(r < ECH)
                def _():
                    pltpu.make_async_copy(
                        table_hbm.at[idx_v.at[j]], rows_v.at[b], gsem.at[b]
                    ).wait()
                    base = pl.multiple_of(r * CHUNK, CHUNK)
                    pltpu.async_copy(
                        rows_v.at[b], out_hbm.at[pl.ds(base, CHUNK)], osem.at[b]
                    )

            for b in range(GNB):
                j = jj + b
                r = wid + NW * j

                @pl.when(r < ECH)
                def _():
                    base = pl.multiple_of(r * CHUNK, CHUNK)
                    pltpu.make_async_copy(
                        rows_v.at[b], out_hbm.at[pl.ds(base, CHUNK)], osem.at[b]
                    ).wait()

    return k(table, idx)


def _sc_segsum_partials(rows, idx, zeros):
    @functools.partial(
        pl.kernel,
        mesh=_sc_mesh(),
        out_type=jax.ShapeDtypeStruct((SC_CORES, N, H), jnp.float32),
        scratch_types=[
            pltpu.VMEM((NCH, CHUNK), jnp.int32),
            pltpu.VMEM((SNB, CHUNK, H), jnp.float32),
            pltpu.SemaphoreType.DMA((SNB,)),
            pltpu.VMEM_SHARED((N, H), jnp.float32),
        ],
    )
    def k(rows_hbm, idx_hbm, zeros_hbm, out_hbm, idx_v, rows_v, rsem, acc):
        cid = lax.axis_index("c")
        sid = lax.axis_index("s")
        wid = sid * SC_CORES + cid
        nbase = pl.multiple_of(sid * NPT, 8)
        pltpu.sync_copy(idx_hbm.at[wid], idx_v)

        @pl.when(sid < 15)
        def _():
            pltpu.sync_copy(
                zeros_hbm.at[pl.ds(nbase, NPT)], acc.at[pl.ds(nbase, NPT)]
            )

        @pl.when(sid == 15)
        def _():
            pltpu.sync_copy(
                zeros_hbm.at[pl.ds(15 * NPT, NPT_LAST)],
                acc.at[pl.ds(15 * NPT, NPT_LAST)],
            )

        plsc.subcore_barrier()

        @pl.loop(0, SPAD, step=SNB)
        def _(jj):
            for b in range(SNB):
                j = jj + b
                r = wid + NW * j

                @pl.when(r < ECH)
                def _():
                    base = pl.multiple_of(r * CHUNK, CHUNK)
                    pltpu.async_copy(
                        rows_hbm.at[pl.ds(base, CHUNK)], rows_v.at[b], rsem.at[b]
                    )

            for b in range(SNB):
                j = jj + b
                r = wid + NW * j

                @pl.when(r < ECH)
                def _():
                    base = pl.multiple_of(r * CHUNK, CHUNK)
                    pltpu.make_async_copy(
                        rows_hbm.at[pl.ds(base, CHUNK)], rows_v.at[b], rsem.at[b]
                    ).wait()
                    pltpu.sync_copy(rows_v.at[b], acc.at[idx_v.at[j]], add=True)

        plsc.subcore_barrier()

        @pl.when(sid < 15)
        def _():
            pltpu.sync_copy(
                acc.at[pl.ds(nbase, NPT)], out_hbm.at[cid].at[pl.ds(nbase, NPT)]
            )

        @pl.when(sid == 15)
        def _():
            pltpu.sync_copy(
                acc.at[pl.ds(15 * NPT, NPT_LAST)],
                out_hbm.at[cid].at[pl.ds(15 * NPT, NPT_LAST)],
            )

    return k(rows, idx, zeros)


def _gather_rows(table, idx):
    return _sc_gather(table, idx)


def _segsum(rows, dst, zeros):
    return _add2(_sc_segsum_partials(rows, dst, zeros))


def kernel(x, edge_index, edge_attr, batch, W1, W2, W3):
    def _arrange(ix):
        ixp = jnp.concatenate(
            [ix.astype(jnp.int32), jnp.zeros((EIDX_PAD - E,), jnp.int32)]
        )
        return ixp.reshape(NCH, NW, CHUNK).transpose(1, 0, 2)

    src1 = _arrange(edge_index[0])
    dst1 = _arrange(edge_index[1])
    w1a, w1b = W1[:DF], W1[DF:]
    w3a, w3b = W3[:DF], W3[DF:]
    bcol = batch.astype(jnp.int32).reshape(N, 1)
    zeros = jnp.zeros((N, H), jnp.float32)

    xa, xw3 = _node_mm(x, w1a, w3a)
    gx = _gather_rows(xa, src1)
    h0, hw0 = _edge_init(gx, edge_attr, w1b, W2)

    agg1 = _segsum(hw0, dst1, zeros)
    g1 = _gather_rows(agg1, src1)
    hw1 = _edge_round(h0, g1, hw0, W2)

    agg2 = _segsum(hw1, dst1, zeros)
    g2 = _gather_rows(agg2, src1)
    h2 = _edge_final(h0, g2, hw1)

    vmsg = _segsum(h2, dst1, zeros)
    return _node_out(xw3, vmsg, w3b, bcol)


# speedup vs baseline: 1.0892x; 1.0051x over previous
import functools

import jax
import jax.numpy as jnp
from jax import lax
from jax.experimental import pallas as pl
from jax.experimental.pallas import tpu as pltpu
from jax.experimental.pallas import tpu_sc as plsc

N = 10000
E = 320000
DF = 128
DE = 16
H = 128
G = 64

EB = 2000
NB = 2000


def _node_mm_body(x_ref, w1a_ref, w3a_ref, xa_ref, xw3_ref):
    x = x_ref[...]
    xa_ref[...] = jnp.dot(x, w1a_ref[...], preferred_element_type=jnp.float32)
    xw3_ref[...] = jnp.dot(x, w3a_ref[...], preferred_element_type=jnp.float32)


def _node_mm(x, w1a, w3a):
    return pl.pallas_call(
        _node_mm_body,
        grid=(N // NB,),
        in_specs=[
            pl.BlockSpec((NB, DF), lambda i: (i, 0)),
            pl.BlockSpec((DF, H), lambda i: (0, 0)),
            pl.BlockSpec((DF, H), lambda i: (0, 0)),
        ],
        out_specs=[
            pl.BlockSpec((NB, H), lambda i: (i, 0)),
            pl.BlockSpec((NB, H), lambda i: (i, 0)),
        ],
        out_shape=[
            jax.ShapeDtypeStruct((N, H), jnp.float32),
            jax.ShapeDtypeStruct((N, H), jnp.float32),
        ],
    )(x, w1a, w3a)


def _edge_init_body(gx_ref, ea_ref, w1b_ref, w2_ref, h0_ref, hw_ref):
    h0 = jax.nn.relu(
        gx_ref[...]
        + jnp.dot(ea_ref[...], w1b_ref[...], preferred_element_type=jnp.float32)
    )
    h0_ref[...] = h0.astype(jnp.bfloat16)
    hw_ref[...] = jnp.dot(
        h0.astype(jnp.bfloat16),
        w2_ref[...].astype(jnp.bfloat16),
        preferred_element_type=jnp.float32,
    )


def _edge_init(gx, ea, w1b, w2):
    return pl.pallas_call(
        _edge_init_body,
        grid=(E // EB,),
        in_specs=[
            pl.BlockSpec((EB, H), lambda i: (i, 0)),
            pl.BlockSpec((EB, DE), lambda i: (i, 0)),
            pl.BlockSpec((DE, H), lambda i: (0, 0)),
            pl.BlockSpec((H, H), lambda i: (0, 0)),
        ],
        out_specs=[
            pl.BlockSpec((EB, H), lambda i: (i, 0)),
            pl.BlockSpec((EB, H), lambda i: (i, 0)),
        ],
        out_shape=[
            jax.ShapeDtypeStruct((E, H), jnp.bfloat16),
            jax.ShapeDtypeStruct((E, H), jnp.float32),
        ],
    )(gx, ea, w1b, w2)


def _pair_swap(hw):
    rows = jax.lax.broadcasted_iota(jnp.int32, hw.shape, 0)
    even = (rows % 2) == 0
    return jnp.where(even, jnp.roll(hw, -1, axis=0), jnp.roll(hw, 1, axis=0))


def _edge_round_body(h0_ref, g_ref, hw_ref, w2_ref, hwn_ref):
    h = jax.nn.relu(
        h0_ref[...].astype(jnp.float32) + g_ref[...] - _pair_swap(hw_ref[...])
    )
    hwn_ref[...] = jnp.dot(
        h.astype(jnp.bfloat16),
        w2_ref[...].astype(jnp.bfloat16),
        preferred_element_type=jnp.float32,
    )


def _edge_round(h0, g, hw, w2):
    return pl.pallas_call(
        _edge_round_body,
        grid=(E // EB,),
        in_specs=[
            pl.BlockSpec((EB, H), lambda i: (i, 0)),
            pl.BlockSpec((EB, H), lambda i: (i, 0)),
            pl.BlockSpec((EB, H), lambda i: (i, 0)),
            pl.BlockSpec((H, H), lambda i: (0, 0)),
        ],
        out_specs=pl.BlockSpec((EB, H), lambda i: (i, 0)),
        out_shape=jax.ShapeDtypeStruct((E, H), jnp.float32),
    )(h0, g, hw, w2)


def _edge_final_body(h0_ref, g_ref, hw_ref, h_ref):
    h_ref[...] = jax.nn.relu(
        h0_ref[...].astype(jnp.float32) + g_ref[...] - _pair_swap(hw_ref[...])
    )


def _edge_final(h0, g, hw):
    return pl.pallas_call(
        _edge_final_body,
        grid=(E // EB,),
        in_specs=[
            pl.BlockSpec((EB, H), lambda i: (i, 0)),
            pl.BlockSpec((EB, H), lambda i: (i, 0)),
            pl.BlockSpec((EB, H), lambda i: (i, 0)),
        ],
        out_specs=pl.BlockSpec((EB, H), lambda i: (i, 0)),
        out_shape=jax.ShapeDtypeStruct((E, H), jnp.float32),
    )(h0, g, hw)


def _node_out_body(xw3_ref, vm_ref, w3b_ref, bcol_ref, out_ref):
    i = pl.program_id(0)

    @pl.when(i == 0)
    def _():
        out_ref[...] = jnp.zeros_like(out_ref)

    na = jax.nn.relu(
        xw3_ref[...]
        + jnp.dot(vm_ref[...], w3b_ref[...], preferred_element_type=jnp.float32)
    )
    gids = jax.lax.broadcasted_iota(jnp.int32, (NB, G), 1)
    onehot = (bcol_ref[...] == gids).astype(jnp.float32)
    out_ref[...] += jax.lax.dot_general(
        onehot, na, (((0,), (0,)), ((), ())), preferred_element_type=jnp.float32
    )


def _node_out(xw3, vmsg, w3b, bcol):
    return pl.pallas_call(
        _node_out_body,
        grid=(N // NB,),
        in_specs=[
            pl.BlockSpec((NB, H), lambda i: (i, 0)),
            pl.BlockSpec((NB, H), lambda i: (i, 0)),
            pl.BlockSpec((H, H), lambda i: (0, 0)),
            pl.BlockSpec((NB, 1), lambda i: (i, 0)),
        ],
        out_specs=pl.BlockSpec((G, H), lambda i: (0, 0)),
        out_shape=jax.ShapeDtypeStruct((G, H), jnp.float32),
        compiler_params=pltpu.CompilerParams(
            dimension_semantics=("arbitrary",)
        ),
    )(xw3, vmsg, w3b, bcol)


def _add2_body(a_ref, b_ref, o_ref):
    o_ref[...] = a_ref[...] + b_ref[...]


def _add2(p):
    return pl.pallas_call(
        _add2_body,
        grid=(N // NB,),
        in_specs=[
            pl.BlockSpec((NB, H), lambda i: (i, 0)),
            pl.BlockSpec((NB, H), lambda i: (i, 0)),
        ],
        out_specs=pl.BlockSpec((NB, H), lambda i: (i, 0)),
        out_shape=jax.ShapeDtypeStruct((N, H), jnp.float32),
    )(p[0], p[1])


SC_CORES = 2
SC_TILES = 16
NW = SC_CORES * SC_TILES
CHUNK = 128
ECH = E // CHUNK
NCH = 80
ECH_PAD = NCH * NW
EIDX_PAD = ECH_PAD * CHUNK
NPT = 632
NPT_LAST = N - 15 * NPT

GNB = 7
SNB = 2
GPAD = ((NCH + GNB - 1) // GNB) * GNB
SPAD = ((NCH + SNB - 1) // SNB) * SNB


def _sc_mesh():
    return plsc.VectorSubcoreMesh(core_axis_name="c", subcore_axis_name="s")


def _sc_gather(table, idx):
    @functools.partial(
        pl.kernel,
        mesh=_sc_mesh(),
        out_type=jax.ShapeDtypeStruct((E, H), jnp.float32),
        scratch_types=[
            pltpu.VMEM((NCH, CHUNK), jnp.int32),
            pltpu.VMEM((GNB, CHUNK, H), jnp.float32),
            pltpu.SemaphoreType.DMA((GNB,)),
            pltpu.SemaphoreType.DMA((GNB,)),
        ],
    )
    def k(table_hbm, idx_hbm, out_hbm, idx_v, rows_v, gsem, osem):
        wid = lax.axis_index("s") * SC_CORES + lax.axis_index("c")
        pltpu.sync_copy(idx_hbm.at[wid], idx_v)

        @pl.loop(0, GPAD, step=GNB)
        def _(jj):
            for b in range(GNB):
                j = jj + b
                r = wid + NW * j

                @pl.when(r < ECH)
                def _():
                    pltpu.async_copy(
                        table_hbm.at[idx_v.at[j]], rows_v.at[b], gsem.at[b]
                    )

            for b in range(GNB):
                j = jj + b
                r = wid + NW * j

                @pl.when(r < ECH)
                def _():
                    pltpu.make_async_copy(
                        table_hbm.at[idx_v.at[j]], rows_v.at[b], gsem.at[b]
                    ).wait()
                    base = pl.multiple_of(r * CHUNK, CHUNK)
                    pltpu.async_copy(
                        rows_v.at[b], out_hbm.at[pl.ds(base, CHUNK)], osem.at[b]
                    )

            for b in range(GNB):
                j = jj + b
                r = wid + NW * j

                @pl.when(r < ECH)
                def _():
                    base = pl.multiple_of(r * CHUNK, CHUNK)
                    pltpu.make_async_copy(
                        rows_v.at[b], out_hbm.at[pl.ds(base, CHUNK)], osem.at[b]
                    ).wait()

    return k(table, idx)


def _sc_segsum_partials(rows, idx, zeros):
    @functools.partial(
        pl.kernel,
        mesh=_sc_mesh(),
        out_type=jax.ShapeDtypeStruct((SC_CORES, N, H), jnp.float32),
        scratch_types=[
            pltpu.VMEM((NCH, CHUNK), jnp.int32),
            pltpu.VMEM((SNB, CHUNK, H), jnp.float32),
            pltpu.SemaphoreType.DMA((SNB,)),
            pltpu.SemaphoreType.DMA((SNB,)),
            pltpu.VMEM_SHARED((N, H), jnp.float32),
        ],
    )
    def k(rows_hbm, idx_hbm, zeros_hbm, out_hbm, idx_v, rows_v, rsem, asem, acc):
        cid = lax.axis_index("c")
        sid = lax.axis_index("s")
        wid = sid * SC_CORES + cid
        nbase = pl.multiple_of(sid * NPT, 8)
        pltpu.sync_copy(idx_hbm.at[wid], idx_v)

        @pl.when(sid < 15)
        def _():
            pltpu.sync_copy(
                zeros_hbm.at[pl.ds(nbase, NPT)], acc.at[pl.ds(nbase, NPT)]
            )

        @pl.when(sid == 15)
        def _():
            pltpu.sync_copy(
                zeros_hbm.at[pl.ds(15 * NPT, NPT_LAST)],
                acc.at[pl.ds(15 * NPT, NPT_LAST)],
            )

        plsc.subcore_barrier()

        @pl.loop(0, SPAD, step=SNB)
        def _(jj):
            for b in range(SNB):
                j = jj + b
                r = wid + NW * j

                @pl.when(r < ECH)
                def _():
                    base = pl.multiple_of(r * CHUNK, CHUNK)
                    pltpu.async_copy(
                        rows_hbm.at[pl.ds(base, CHUNK)], rows_v.at[b], rsem.at[b]
                    )

            for b in range(SNB):
                j = jj + b
                r = wid + NW * j

                @pl.when(r < ECH)
                def _():
                    base = pl.multiple_of(r * CHUNK, CHUNK)
                    pltpu.make_async_copy(
                        rows_hbm.at[pl.ds(base, CHUNK)], rows_v.at[b], rsem.at[b]
                    ).wait()
                    pltpu.async_copy(
                        rows_v.at[b], acc.at[idx_v.at[j]], asem.at[b], add=True
                    )

            for b in range(SNB):
                j = jj + b
                r = wid + NW * j

                @pl.when(r < ECH)
                def _():
                    pltpu.make_async_copy(
                        rows_v.at[b], acc.at[idx_v.at[j]], asem.at[b]
                    ).wait()

        plsc.subcore_barrier()

        @pl.when(sid < 15)
        def _():
            pltpu.sync_copy(
                acc.at[pl.ds(nbase, NPT)], out_hbm.at[cid].at[pl.ds(nbase, NPT)]
            )

        @pl.when(sid == 15)
        def _():
            pltpu.sync_copy(
                acc.at[pl.ds(15 * NPT, NPT_LAST)],
                out_hbm.at[cid].at[pl.ds(15 * NPT, NPT_LAST)],
            )

    return k(rows, idx, zeros)


def _gather_rows(table, idx):
    return _sc_gather(table, idx)


def _segsum(rows, dst, zeros):
    return _add2(_sc_segsum_partials(rows, dst, zeros))


def kernel(x, edge_index, edge_attr, batch, W1, W2, W3):
    def _arrange(ix):
        ixp = jnp.concatenate(
            [ix.astype(jnp.int32), jnp.zeros((EIDX_PAD - E,), jnp.int32)]
        )
        return ixp.reshape(NCH, NW, CHUNK).transpose(1, 0, 2)

    src1 = _arrange(edge_index[0])
    dst1 = _arrange(edge_index[1])
    w1a, w1b = W1[:DF], W1[DF:]
    w3a, w3b = W3[:DF], W3[DF:]
    bcol = batch.astype(jnp.int32).reshape(N, 1)
    zeros = jnp.zeros((N, H), jnp.float32)

    xa, xw3 = _node_mm(x, w1a, w3a)
    gx = _gather_rows(xa, src1)
    h0, hw0 = _edge_init(gx, edge_attr, w1b, W2)

    agg1 = _segsum(hw0, dst1, zeros)
    g1 = _gather_rows(agg1, src1)
    hw1 = _edge_round(h0, g1, hw0, W2)

    agg2 = _segsum(hw1, dst1, zeros)
    g2 = _gather_rows(agg2, src1)
    h2 = _edge_final(h0, g2, hw1)

    vmsg = _segsum(h2, dst1, zeros)
    return _node_out(xw3, vmsg, w3b, bcol)
